# baseline (speedup 1.0000x reference)
import functools

import jax
import jax.numpy as jnp
from jax import lax
from jax.experimental import pallas as pl
from jax.experimental.pallas import tpu as pltpu
from jax.experimental.pallas import tpu_sc as plsc

A = 4
K = 1024
D = 64
B = 16
T = 576
N = B * T

NC = 2
NS = 16
BG = 8


def _tc_body(x_ref, w_ref, idx_ref, loss_ref):
    b = pl.program_id(0)

    for a in range(A):
        x = x_ref[0, :, T * a:T * (a + 1)]
        w = w_ref[a]
        xt = x.T
        mm2 = jax.lax.dot_general(xt + xt, w, (((1,), (1,)), ((), ())),
                                  preferred_element_type=jnp.float32)
        sx = jnp.sum(xt * xt, axis=1, keepdims=True)
        sw = jnp.sum(w * w, axis=1)[None, :]
        dist = (sx + sw) - mm2

        m = jnp.min(dist, axis=1, keepdims=True)
        lane_f = jax.lax.broadcasted_iota(jnp.int32, (T, K), 1).astype(
            jnp.float32)
        idx_f = jnp.min(jnp.where(dist == m, lane_f, jnp.float32(K)),
                        axis=1, keepdims=True)

        idx_ref[0, :, a:a + 1] = idx_f.astype(jnp.int32)
        loss_ref[jax.lax.rem(b, 8), a] = jnp.sum(m)


def _vq_tc(x2, emb):
    return pl.pallas_call(
        _tc_body,
        grid=(B,),
        in_specs=[
            pl.BlockSpec((1, D, A * T), lambda b: (b, 0, 0)),
            pl.BlockSpec((A, K, D), lambda b: (0, 0, 0)),
        ],
        out_specs=[
            pl.BlockSpec((1, T, A), lambda b: (b, 0, 0)),
            pl.BlockSpec((8, A), lambda b: (b // 8, 0),
                         memory_space=pltpu.SMEM),
        ],
        out_shape=[
            jax.ShapeDtypeStruct((B, T, A), jnp.int32),
            jax.ShapeDtypeStruct((B, A), jnp.float32),
        ],
        compiler_params=pltpu.CompilerParams(
            dimension_semantics=("parallel",)),
    )(x2, emb)


def _sc_body(wt_hbm, idx_hbm, q_hbm, histp_hbm, counts_hbm,
             wtbuf, idxblk, qbuf, hist, histin):
    c = lax.axis_index("c")
    s = lax.axis_index("s")
    a = c * 2 + s // BG
    bg = jax.lax.rem(s, BG)

    iota16 = lax.iota(jnp.int32, 16)
    ones16 = jnp.ones((16,), jnp.float32)
    mask16 = iota16 < 16
    zeros16 = jnp.zeros((16,), jnp.float32)

    pltpu.sync_copy(wt_hbm.at[pl.ds(a * (D * K), D * K)], wtbuf)

    def _zero(i, carry):
        hist[pl.ds(i * 16, 16)] = zeros16
        return carry
    lax.fori_loop(0, K // 16, _zero, 0)

    iotaA = iota16 * A + a
    for r in range(2):
        b = bg * 2 + r
        pltpu.sync_copy(idx_hbm.at[b], idxblk)

        def _chunk(cix, carry):
            iv = plsc.load_gather(idxblk, [iotaA + cix * (16 * A)],
                                  mask=mask16)
            for d in range(D):
                vals = plsc.load_gather(wtbuf, [iv + d * K], mask=mask16)
                qbuf[d, pl.ds(cix * 16, 16)] = vals
            plsc.addupdate_scatter(hist, [iv], ones16, mask=mask16)
            return carry
        lax.fori_loop(0, T // 16, _chunk, 0)

        pltpu.sync_copy(qbuf, q_hbm.at[b, :, a, :])

    pltpu.sync_copy(hist, histp_hbm.at[a, pl.ds(bg * K, K)])
    plsc.subcore_barrier()

    @pl.when(bg == 0)
    def _reduce():
        pltpu.sync_copy(histp_hbm.at[a], histin)

        def _rchunk(i, carry):
            acc = histin[pl.ds(i * 16, 16)]
            for rr in range(1, BG):
                acc = acc + histin[pl.ds(rr * K + i * 16, 16)]
            hist[pl.ds(i * 16, 16)] = acc
            return carry
        lax.fori_loop(0, K // 16, _rchunk, 0)
        pltpu.sync_copy(hist, counts_hbm.at[a])


def _vq_sc(wtflat, idx2):
    mesh = plsc.VectorSubcoreMesh(core_axis_name="c", subcore_axis_name="s")
    f = pl.kernel(
        _sc_body,
        mesh=mesh,
        compiler_params=pltpu.CompilerParams(needs_layout_passes=False),
        out_type=[
            jax.ShapeDtypeStruct((B, D, A, T), jnp.float32),
            jax.ShapeDtypeStruct((A, BG * K), jnp.float32),
            jax.ShapeDtypeStruct((A, K), jnp.float32),
        ],
        scratch_types=[
            pltpu.VMEM((D * K,), jnp.float32),
            pltpu.VMEM((T * A,), jnp.int32),
            pltpu.VMEM((D, T), jnp.float32),
            pltpu.VMEM((K,), jnp.float32),
            pltpu.VMEM((BG * K,), jnp.float32),
        ],
    )
    return f(wtflat, idx2)


def kernel(inputs, emb):
    x2 = inputs.reshape(B, D, A * T)
    wtflat = jnp.transpose(emb, (0, 2, 1)).reshape(A * D * K)
    idx2, loss_sums = _vq_tc(x2, emb)
    quantized, _histp, counts = _vq_sc(wtflat, idx2.reshape(B, T * A))
    encoding_indices = idx2.reshape(N, A, 1)
    l = jnp.sum(loss_sums, axis=0) / jnp.float32(N * D)
    q_loss = jnp.sum(l) / A
    e_loss = jnp.sum(0.25 * l) / A
    p = counts / N
    perplexity = jnp.sum(jnp.exp(-jnp.sum(p * jnp.log(p + 1e-10), axis=1))) / A
    return q_loss, e_loss, quantized, perplexity, encoding_indices


# speedup vs baseline: 1.0016x; 1.0016x over previous
import functools

import jax
import jax.numpy as jnp
from jax import lax
from jax.experimental import pallas as pl
from jax.experimental.pallas import tpu as pltpu
from jax.experimental.pallas import tpu_sc as plsc

A = 4
K = 1024
D = 64
B = 16
T = 576
N = B * T

NC = 2
NS = 16
BG = 8


def _tc_body(x_ref, w_ref, idx_ref, loss_ref):
    b = pl.program_id(0)

    for a in range(A):
        x = x_ref[0, :, T * a:T * (a + 1)]
        w = w_ref[a]
        xt = x.T
        mm2 = jax.lax.dot_general(xt + xt, w, (((1,), (1,)), ((), ())),
                                  preferred_element_type=jnp.float32)
        sx = jnp.sum(xt * xt, axis=1, keepdims=True)
        sw = jnp.sum(w * w, axis=1)[None, :]
        dist = (sx + sw) - mm2

        m = jnp.min(dist, axis=1, keepdims=True)
        lane_f = jax.lax.broadcasted_iota(jnp.int32, (T, K), 1).astype(
            jnp.float32)
        idx_f = jnp.min(jnp.where(dist == m, lane_f, jnp.float32(K)),
                        axis=1, keepdims=True)

        idx_ref[0, :, a:a + 1] = idx_f.astype(jnp.int32)
        loss_ref[jax.lax.rem(b, 8), a] = jnp.sum(m)


def _vq_tc(x2, emb):
    return pl.pallas_call(
        _tc_body,
        grid=(B,),
        in_specs=[
            pl.BlockSpec((1, D, A * T), lambda b: (b, 0, 0)),
            pl.BlockSpec((A, K, D), lambda b: (0, 0, 0)),
        ],
        out_specs=[
            pl.BlockSpec((1, T, A), lambda b: (b, 0, 0)),
            pl.BlockSpec((8, A), lambda b: (b // 8, 0),
                         memory_space=pltpu.SMEM),
        ],
        out_shape=[
            jax.ShapeDtypeStruct((B, T, A), jnp.int32),
            jax.ShapeDtypeStruct((B, A), jnp.float32),
        ],
        compiler_params=pltpu.CompilerParams(
            dimension_semantics=("arbitrary",)),
    )(x2, emb)


def _sc_body(wt_hbm, idx_hbm, q_hbm, histp_hbm, counts_hbm,
             wtbuf, idxblk, qbuf, hist, histin):
    c = lax.axis_index("c")
    s = lax.axis_index("s")
    a = c * 2 + s // BG
    bg = jax.lax.rem(s, BG)

    iota16 = lax.iota(jnp.int32, 16)
    ones16 = jnp.ones((16,), jnp.float32)
    mask16 = iota16 < 16
    zeros16 = jnp.zeros((16,), jnp.float32)

    pltpu.sync_copy(wt_hbm.at[pl.ds(a * (D * K), D * K)], wtbuf)

    def _zero(i, carry):
        hist[pl.ds(i * 16, 16)] = zeros16
        return carry
    lax.fori_loop(0, K // 16, _zero, 0)

    iotaA = iota16 * A + a
    for r in range(2):
        b = bg * 2 + r
        pltpu.sync_copy(idx_hbm.at[b], idxblk)

        def _chunk(cix, carry):
            iv = plsc.load_gather(idxblk, [iotaA + cix * (16 * A)],
                                  mask=mask16)
            for d in range(D):
                vals = plsc.load_gather(wtbuf, [iv + d * K], mask=mask16)
                qbuf[d, pl.ds(cix * 16, 16)] = vals
            plsc.addupdate_scatter(hist, [iv], ones16, mask=mask16)
            return carry
        lax.fori_loop(0, T // 16, _chunk, 0)

        pltpu.sync_copy(qbuf, q_hbm.at[b, :, a, :])

    pltpu.sync_copy(hist, histp_hbm.at[a, pl.ds(bg * K, K)])
    plsc.subcore_barrier()

    @pl.when(bg == 0)
    def _reduce():
        pltpu.sync_copy(histp_hbm.at[a], histin)

        def _rchunk(i, carry):
            acc = histin[pl.ds(i * 16, 16)]
            for rr in range(1, BG):
                acc = acc + histin[pl.ds(rr * K + i * 16, 16)]
            hist[pl.ds(i * 16, 16)] = acc
            return carry
        lax.fori_loop(0, K // 16, _rchunk, 0)
        pltpu.sync_copy(hist, counts_hbm.at[a])


def _vq_sc(wtflat, idx2):
    mesh = plsc.VectorSubcoreMesh(core_axis_name="c", subcore_axis_name="s")
    f = pl.kernel(
        _sc_body,
        mesh=mesh,
        compiler_params=pltpu.CompilerParams(needs_layout_passes=False),
        out_type=[
            jax.ShapeDtypeStruct((B, D, A, T), jnp.float32),
            jax.ShapeDtypeStruct((A, BG * K), jnp.float32),
            jax.ShapeDtypeStruct((A, K), jnp.float32),
        ],
        scratch_types=[
            pltpu.VMEM((D * K,), jnp.float32),
            pltpu.VMEM((T * A,), jnp.int32),
            pltpu.VMEM((D, T), jnp.float32),
            pltpu.VMEM((K,), jnp.float32),
            pltpu.VMEM((BG * K,), jnp.float32),
        ],
    )
    return f(wtflat, idx2)


def kernel(inputs, emb):
    x2 = inputs.reshape(B, D, A * T)
    wtflat = jnp.transpose(emb, (0, 2, 1)).reshape(A * D * K)
    idx2, loss_sums = _vq_tc(x2, emb)
    quantized, _histp, counts = _vq_sc(wtflat, idx2.reshape(B, T * A))
    encoding_indices = idx2.reshape(N, A, 1)
    l = jnp.sum(loss_sums, axis=0) / jnp.float32(N * D)
    q_loss = jnp.sum(l) / A
    e_loss = jnp.sum(0.25 * l) / A
    p = counts / N
    perplexity = jnp.sum(jnp.exp(-jnp.sum(p * jnp.log(p + 1e-10), axis=1))) / A
    return q_loss, e_loss, quantized, perplexity, encoding_indices


# speedup vs baseline: 1.1254x; 1.1236x over previous
import functools

import jax
import jax.numpy as jnp
from jax import lax
from jax.experimental import pallas as pl
from jax.experimental.pallas import tpu as pltpu
from jax.experimental.pallas import tpu_sc as plsc

A = 4
K = 1024
D = 64
B = 16
T = 576
N = B * T

NC = 2
NS = 16
BG = 8


def _tc_body(x_ref, w_ref, idx_ref, loss_ref, sw_ref):
    b = pl.program_id(0)

    @pl.when(b == 0)
    def _reset():
        for a in range(A):
            loss_ref[a] = 0.0
            w = w_ref[a]
            sw_ref[a:a + 1, :] = jnp.sum(w * w, axis=1)[None, :]

    for a in range(A):
        x = x_ref[0, :, T * a:T * (a + 1)]
        w = w_ref[a]
        xt = x.T
        mm2 = jax.lax.dot_general(xt + xt, w, (((1,), (1,)), ((), ())),
                                  preferred_element_type=jnp.float32)
        sx = jnp.sum(xt * xt, axis=1, keepdims=True)
        sw = sw_ref[a:a + 1, :]
        dist = (sx + sw) - mm2

        m = jnp.min(dist, axis=1, keepdims=True)
        lane_f = jax.lax.broadcasted_iota(jnp.int32, (T, K), 1).astype(
            jnp.float32)
        idx_f = jnp.min(jnp.where(dist == m, lane_f, jnp.float32(K)),
                        axis=1, keepdims=True)

        idx_ref[0, :, a:a + 1] = idx_f.astype(jnp.int32)
        loss_ref[a] += jnp.sum(m)


def _vq_tc(x2, emb):
    return pl.pallas_call(
        _tc_body,
        grid=(B,),
        in_specs=[
            pl.BlockSpec((1, D, A * T), lambda b: (b, 0, 0)),
            pl.BlockSpec((A, K, D), lambda b: (0, 0, 0)),
        ],
        out_specs=[
            pl.BlockSpec((1, T, A), lambda b: (b, 0, 0)),
            pl.BlockSpec(memory_space=pltpu.SMEM),
        ],
        out_shape=[
            jax.ShapeDtypeStruct((B, T, A), jnp.int32),
            jax.ShapeDtypeStruct((A,), jnp.float32),
        ],
        scratch_shapes=[
            pltpu.VMEM((A, K), jnp.float32),
        ],
    )(x2, emb)


def _sc_body(wt_hbm, idx_hbm, q_hbm, histp_hbm, counts_hbm,
             wtbuf, idxblk, qbuf, hist, histin):
    c = lax.axis_index("c")
    s = lax.axis_index("s")
    a = c * 2 + s // BG
    bg = jax.lax.rem(s, BG)

    iota16 = lax.iota(jnp.int32, 16)
    ones16 = jnp.ones((16,), jnp.float32)
    mask16 = iota16 < 16
    zeros16 = jnp.zeros((16,), jnp.float32)

    pltpu.sync_copy(wt_hbm.at[pl.ds(a * (D * K), D * K)], wtbuf)

    def _zero(i, carry):
        hist[pl.ds(i * 16, 16)] = zeros16
        return carry
    lax.fori_loop(0, K // 16, _zero, 0)

    iotaA = iota16 * A + a
    for r in range(2):
        b = bg * 2 + r
        pltpu.sync_copy(idx_hbm.at[b], idxblk)

        def _chunk(cix, carry):
            iv = plsc.load_gather(idxblk, [iotaA + cix * (16 * A)],
                                  mask=mask16)
            for d in range(D):
                vals = plsc.load_gather(wtbuf, [iv + d * K], mask=mask16)
                qbuf[d, pl.ds(cix * 16, 16)] = vals
            plsc.addupdate_scatter(hist, [iv], ones16, mask=mask16)
            return carry
        lax.fori_loop(0, T // 16, _chunk, 0)

        pltpu.sync_copy(qbuf, q_hbm.at[b, :, a, :])

    pltpu.sync_copy(hist, histp_hbm.at[a, pl.ds(bg * K, K)])
    plsc.subcore_barrier()

    @pl.when(bg == 0)
    def _reduce():
        pltpu.sync_copy(histp_hbm.at[a], histin)

        def _rchunk(i, carry):
            acc = histin[pl.ds(i * 16, 16)]
            for rr in range(1, BG):
                acc = acc + histin[pl.ds(rr * K + i * 16, 16)]
            hist[pl.ds(i * 16, 16)] = acc
            return carry
        lax.fori_loop(0, K // 16, _rchunk, 0)
        pltpu.sync_copy(hist, counts_hbm.at[a])


def _vq_sc(wtflat, idx2):
    mesh = plsc.VectorSubcoreMesh(core_axis_name="c", subcore_axis_name="s")
    f = pl.kernel(
        _sc_body,
        mesh=mesh,
        compiler_params=pltpu.CompilerParams(needs_layout_passes=False),
        out_type=[
            jax.ShapeDtypeStruct((B, D, A, T), jnp.float32),
            jax.ShapeDtypeStruct((A, BG * K), jnp.float32),
            jax.ShapeDtypeStruct((A, K), jnp.float32),
        ],
        scratch_types=[
            pltpu.VMEM((D * K,), jnp.float32),
            pltpu.VMEM((T * A,), jnp.int32),
            pltpu.VMEM((D, T), jnp.float32),
            pltpu.VMEM((K,), jnp.float32),
            pltpu.VMEM((BG * K,), jnp.float32),
        ],
    )
    return f(wtflat, idx2)


def kernel(inputs, emb):
    x2 = inputs.reshape(B, D, A * T)
    wtflat = jnp.transpose(emb, (0, 2, 1)).reshape(A * D * K)
    idx2, loss_sums = _vq_tc(x2, emb)
    quantized, _histp, counts = _vq_sc(wtflat, idx2.reshape(B, T * A))
    encoding_indices = idx2.reshape(N, A, 1)
    l = loss_sums / jnp.float32(N * D)
    q_loss = jnp.sum(l) / A
    e_loss = jnp.sum(0.25 * l) / A
    p = counts / N
    perplexity = jnp.sum(jnp.exp(-jnp.sum(p * jnp.log(p + 1e-10), axis=1))) / A
    return q_loss, e_loss, quantized, perplexity, encoding_indices
